# baseline (speedup 1.0000x reference)
import functools

import jax
import jax.numpy as jnp
from jax import lax
from jax.experimental import pallas as pl
from jax.experimental.pallas import tpu as pltpu
from jax.experimental.pallas import tpu_sc as plsc

N, C = 16384, 1000
NC, NS, L = 2, 16, 16
NW = NC * NS
RPW = N // NW
NST = RPW // L


def _sc_partial_sums(prob, target, reward):
    mesh = plsc.VectorSubcoreMesh(core_axis_name="c", subcore_axis_name="s")

    @functools.partial(
        pl.kernel,
        out_type=jax.ShapeDtypeStruct((NW * L,), jnp.float32),
        mesh=mesh,
        compiler_params=pltpu.CompilerParams(needs_layout_passes=False),
        scratch_types=[
            pltpu.VMEM((RPW,), jnp.int32),
            pltpu.VMEM((RPW,), jnp.float32),
            pltpu.VMEM((L, 896), jnp.float32),
            pltpu.VMEM((L, 896), jnp.float32),
            pltpu.VMEM((L, 104), jnp.float32),
            pltpu.VMEM((L, 104), jnp.float32),
            pltpu.VMEM((L,), jnp.float32),
            pltpu.SemaphoreType.DMA,
            pltpu.SemaphoreType.DMA,
        ],
    )
    def k(prob_hbm, tgt_hbm, rew_hbm, out_hbm, tgt_v, rew_v, buf0, buf1,
          ebuf0, ebuf1, acc_v, sem0, sem1):
        wid = lax.axis_index("s") * NC + lax.axis_index("c")
        base = wid * RPW
        pltpu.sync_copy(tgt_hbm.at[pl.ds(base, RPW)], tgt_v)
        pltpu.sync_copy(rew_hbm.at[pl.ds(base, RPW)], rew_v)

        bufs = (buf0, buf1)
        ebufs = (ebuf0, ebuf1)
        sems = (sem0, sem1)

        def fetch(k_):
            r0 = pl.multiple_of(base + k_ * L, 8)
            cp = pltpu.async_copy(
                prob_hbm.at[pl.ds(r0, L), pl.ds(0, 896)],
                bufs[k_ % 2], sems[k_ % 2])
            pltpu.async_copy(
                prob_hbm.at[pl.ds(r0, L), pl.ds(896, 104)],
                ebufs[k_ % 2], sems[k_ % 2])
            return cp

        lane = lax.broadcasted_iota(jnp.int32, (L,), 0)
        copies = [None, None]
        copies[0] = fetch(0)
        acc = jnp.zeros((L,), jnp.float32)
        for k_ in range(NST):
            if k_ + 1 < NST:
                copies[(k_ + 1) % 2] = fetch(k_ + 1)
            copies[k_ % 2].wait()
            pltpu.make_async_copy(
                prob_hbm.at[pl.ds(0, L), pl.ds(896, 104)],
                ebufs[k_ % 2], sems[k_ % 2]).wait()
            t = tgt_v[pl.ds(k_ * L, L)]
            va = plsc.load_gather(bufs[k_ % 2], [lane, jnp.minimum(t, 895)])
            vb = plsc.load_gather(ebufs[k_ % 2],
                                  [lane, jnp.maximum(t - 896, 0)])
            vals = jnp.where(t < 896, va, vb)
            acc = acc + vals * rew_v[pl.ds(k_ * L, L)]
        acc_v[...] = acc
        pltpu.sync_copy(acc_v, out_hbm.at[pl.ds(wid * L, L)])

    return k(prob, target, reward)


def kernel(prob, target, reward, device):
    partials = _sc_partial_sums(prob, target, reward)
    return -jnp.sum(partials) * (1.0 / N)


# speedup vs baseline: 1.0914x; 1.0914x over previous
import functools

import jax
import jax.numpy as jnp
from jax import lax
from jax.experimental import pallas as pl
from jax.experimental.pallas import tpu as pltpu
from jax.experimental.pallas import tpu_sc as plsc

N, C = 16384, 1000
NC, NS, L = 2, 16, 16
NW = NC * NS

N_TC = 10240
N_SC = N - N_TC
RPW = N_SC // NW
NST = RPW // L

BR = 1024
G_TC = N_TC // BR
TV = 128
SUB = BR // TV


def _sc_partial_sums(prob, target, reward):
    mesh = plsc.VectorSubcoreMesh(core_axis_name="c", subcore_axis_name="s")

    @functools.partial(
        pl.kernel,
        out_type=jax.ShapeDtypeStruct((NW * L,), jnp.float32),
        mesh=mesh,
        compiler_params=pltpu.CompilerParams(needs_layout_passes=False),
        scratch_types=[
            pltpu.VMEM((RPW,), jnp.int32),
            pltpu.VMEM((RPW,), jnp.float32),
            pltpu.VMEM((L, C), jnp.float32),
            pltpu.VMEM((L, C), jnp.float32),
            pltpu.VMEM((L,), jnp.float32),
            pltpu.SemaphoreType.DMA,
            pltpu.SemaphoreType.DMA,
        ],
    )
    def k(prob_hbm, tgt_hbm, rew_hbm, out_hbm, tgt_v, rew_v, buf0, buf1,
          acc_v, sem0, sem1):
        wid = lax.axis_index("s") * NC + lax.axis_index("c")
        base = N_TC + wid * RPW
        pltpu.sync_copy(tgt_hbm.at[pl.ds(base, RPW)], tgt_v)
        pltpu.sync_copy(rew_hbm.at[pl.ds(base, RPW)], rew_v)

        bufs = (buf0, buf1)
        sems = (sem0, sem1)

        def fetch(k_):
            return pltpu.async_copy(
                prob_hbm.at[pl.ds(pl.multiple_of(base + k_ * L, 8), L), :],
                bufs[k_ % 2], sems[k_ % 2])

        lane = lax.broadcasted_iota(jnp.int32, (L,), 0)
        copies = [None, None]
        copies[0] = fetch(0)
        acc = jnp.zeros((L,), jnp.float32)
        for k_ in range(NST):
            if k_ + 1 < NST:
                copies[(k_ + 1) % 2] = fetch(k_ + 1)
            copies[k_ % 2].wait()
            t = tgt_v[pl.ds(k_ * L, L)]
            vals = plsc.load_gather(bufs[k_ % 2], [lane, t])
            acc = acc + vals * rew_v[pl.ds(k_ * L, L)]
        acc_v[...] = acc
        pltpu.sync_copy(acc_v, out_hbm.at[pl.ds(wid * L, L)])

    return k(prob, target, reward)


def _tc_body(t_ref, w_ref, p_ref, o_ref):
    g = pl.program_id(0)
    tT = jnp.transpose(t_ref[...])
    wT = jnp.transpose(w_ref[...])

    part = jnp.zeros((), jnp.float32)
    iota = lax.broadcasted_iota(jnp.int32, (TV, C), 1)
    for a in range(SUB):
        t_col = tT[:, a:a + 1]
        w_col = wT[:, a:a + 1]
        pr = p_ref[pl.ds(a * TV, TV), :]
        tb = jnp.broadcast_to(t_col, (TV, C))
        wb = jnp.broadcast_to(w_col, (TV, C))
        part = part + jnp.sum(jnp.where(tb == iota, pr * wb, 0.0))

    @pl.when(g == 0)
    def _():
        o_ref[0, 0] = 0.0

    o_ref[0, 0] += part


def _tc_partial(prob, target, reward):
    tv = target.reshape(N // TV, TV)
    wv = reward.reshape(N // TV, TV)
    out = pl.pallas_call(
        _tc_body,
        grid=(G_TC,),
        in_specs=[
            pl.BlockSpec((SUB, TV), lambda g: (g, 0)),
            pl.BlockSpec((SUB, TV), lambda g: (g, 0)),
            pl.BlockSpec((BR, C), lambda g: (g, 0)),
        ],
        out_specs=pl.BlockSpec(memory_space=pltpu.SMEM),
        out_shape=jax.ShapeDtypeStruct((1, 1), jnp.float32),
    )(tv, wv, prob)
    return out[0, 0]


def kernel(prob, target, reward, device):
    sc_partials = _sc_partial_sums(prob, target, reward)
    tc_part = _tc_partial(prob, target, reward)
    return -(tc_part + jnp.sum(sc_partials)) * (1.0 / N)
